# baseline (speedup 1.0000x reference)
import jax
import jax.numpy as jnp
from jax import lax
from jax.experimental import pallas as pl
from jax.experimental.pallas import tpu as pltpu
from jax.experimental.pallas import tpu_sc as plsc

N_TOKENS = 262144
TOKEN_DIM = 100
ROW_PAD = 128
OUT_DIM = 3
NW = 32
TOK_PER_W = N_TOKENS // NW
CH = 256
NCH = TOK_PER_W // CH
NGRP = CH // 16
SLICE_STARTS = (0, 16, 32, 48, 64, 80, 84)


def _build_wv(W):
    W_ext = jnp.zeros((OUT_DIM, 112), W.dtype).at[:, :TOKEN_DIM].set(W)
    rows = []
    for k in range(OUT_DIM):
        for i, st in enumerate(SLICE_STARTS):
            v = lax.dynamic_slice(W_ext[k], (st,), (16,))
            if i == 6:
                v = v * (jnp.arange(16) >= 12).astype(W.dtype)
            rows.append(v)
    return jnp.stack(rows)


def _sc_body(p_hbm, wv_hbm, bb_hbm, out_hbm, xbuf0, xbuf1, lgbuf, obuf, wv_v, bb_v, in_sem):
    xbufs = (xbuf0, xbuf1)
    wid = lax.axis_index("s") * 2 + lax.axis_index("c")
    base = wid * TOK_PER_W

    pltpu.sync_copy(wv_hbm, wv_v)
    pltpu.sync_copy(bb_hbm, bb_v)
    wvec = [[wv_v[k * 7 + i, :] for i in range(7)] for k in range(OUT_DIM)]
    bvec = [bb_v[k, :] for k in range(OUT_DIM)]
    iota = lax.iota(jnp.int32, 16)
    iota4 = iota * 4
    lane15 = iota == 15

    def start_in(c, rb):
        pltpu.make_async_copy(
            p_hbm.at[pl.ds(base + c * CH, CH), :],
            xbufs[rb],
            in_sem.at[rb],
        ).start()

    for rb in range(2):
        start_in(rb, rb)

    def chunk_pair(c2, carry):
        for rb in range(2):
            c = c2 * 2 + rb
            xb = xbufs[rb]
            pltpu.make_async_copy(
                p_hbm.at[pl.ds(base + c * CH, CH), :],
                xb,
                in_sem.at[rb],
            ).wait()

            def group(g, carry2):
                t0 = g * 16
                for j in range(16):
                    t = t0 + j
                    xs = [xb[t, pl.ds(st, 16)] for st in SLICE_STARTS]
                    for k in range(OUT_DIM):
                        acc = xs[0] * wvec[k][0]
                        for i in range(1, 7):
                            acc = acc + xs[i] * wvec[k][i]
                        cum = plsc.cumsum(acc)
                        addr = jnp.full((16,), k * CH, jnp.int32) + (t0 + j)
                        plsc.store_scatter(lgbuf, [addr], cum, mask=lane15)
                l0 = lgbuf[pl.ds(t0, 16)] + bvec[0]
                l1 = lgbuf[pl.ds(CH + t0, 16)] + bvec[1]
                l2 = lgbuf[pl.ds(2 * CH + t0, 16)] + bvec[2]
                m = jnp.maximum(jnp.maximum(l0, l1), l2)
                e0 = jnp.exp(l0 - m)
                e1 = jnp.exp(l1 - m)
                e2 = jnp.exp(l2 - m)
                r = 1.0 / (e0 + e1 + e2)
                for k, ek in enumerate((e0, e1, e2)):
                    plsc.store_scatter(obuf, [iota4 + (t0 * 4 + k)], ek * r)
                return carry2

            lax.fori_loop(0, NGRP, group, 0)
            pltpu.sync_copy(
                obuf, out_hbm.at[pl.ds((base + c * CH) * 4, CH * 4)]
            )

            @pl.when(c + 2 < NCH)
            def _():
                start_in(c + 2, rb)
        return carry

    lax.fori_loop(0, NCH // 2, chunk_pair, 0)


def kernel(payload, aux, W, b):
    wv = _build_wv(W)
    bb = jnp.broadcast_to(b[:, None], (OUT_DIM, 16))
    mesh = plsc.VectorSubcoreMesh(core_axis_name="c", subcore_axis_name="s")
    flat = pl.kernel(
        _sc_body,
        out_type=jax.ShapeDtypeStruct((N_TOKENS * 4,), jnp.float32),
        mesh=mesh,
        compiler_params=pltpu.CompilerParams(
            needs_layout_passes=False,
            use_tc_tiling_on_sc=True,
        ),
        scratch_types=[
            pltpu.VMEM((CH, TOKEN_DIM), jnp.float32),
            pltpu.VMEM((CH, TOKEN_DIM), jnp.float32),
            pltpu.VMEM((OUT_DIM * CH,), jnp.float32),
            pltpu.VMEM((CH * 4,), jnp.float32),
            pltpu.VMEM((OUT_DIM * 7, 16), jnp.float32),
            pltpu.VMEM((OUT_DIM, 16), jnp.float32),
            pltpu.SemaphoreType.DMA((2,)),
        ],
    )(payload, wv, bb)
    return flat.reshape(N_TOKENS, 4)[:, :OUT_DIM]


# speedup vs baseline: 1.2198x; 1.2198x over previous
import jax
import jax.numpy as jnp
from jax import lax
from jax.experimental import pallas as pl
from jax.experimental.pallas import tpu as pltpu
from jax.experimental.pallas import tpu_sc as plsc

N_TOKENS = 262144
TOKEN_DIM = 100
ROW_PAD = 128
OUT_DIM = 3
NW = 32
TOK_PER_W = N_TOKENS // NW
CH = 256
NCH = TOK_PER_W // CH
NGRP = CH // 16
SLICE_STARTS = (0, 16, 32, 48, 64, 80, 84)


def _build_wv(W):
    W_ext = jnp.zeros((OUT_DIM, 112), W.dtype).at[:, :TOKEN_DIM].set(W)
    rows = []
    for k in range(OUT_DIM):
        for i, st in enumerate(SLICE_STARTS):
            v = lax.dynamic_slice(W_ext[k], (st,), (16,))
            if i == 6:
                v = v * (jnp.arange(16) >= 12).astype(W.dtype)
            rows.append(v)
    return jnp.stack(rows)


def _sc_body(p_hbm, wv_hbm, bb_hbm, out_hbm, xbuf0, xbuf1, lgbuf, obuf, wv_v, bb_v, in_sem):
    xbufs = (xbuf0, xbuf1)
    wid = lax.axis_index("s") * 2 + lax.axis_index("c")
    base = wid * TOK_PER_W

    pltpu.sync_copy(wv_hbm, wv_v)
    pltpu.sync_copy(bb_hbm, bb_v)
    wvec = [[wv_v[k * 7 + i, :] for i in range(7)] for k in range(OUT_DIM)]
    bvec = [bb_v[k, :] for k in range(OUT_DIM)]
    iota = lax.iota(jnp.int32, 16)
    iota4 = iota * 4
    lane15 = iota == 15

    def start_in(c, rb):
        pltpu.make_async_copy(
            p_hbm.at[pl.ds(base + c * CH, CH), :],
            xbufs[rb],
            in_sem.at[rb],
        ).start()

    for rb in range(2):
        start_in(rb, rb)

    def chunk_pair(c2, carry):
        for rb in range(2):
            c = c2 * 2 + rb
            xb = xbufs[rb]
            pltpu.make_async_copy(
                p_hbm.at[pl.ds(base + c * CH, CH), :],
                xb,
                in_sem.at[rb],
            ).wait()

            def group(g, carry2):
                t0 = g * 16
                for j in range(16):
                    t = t0 + j
                    xs = [xb[t, pl.ds(st, 16)] for st in SLICE_STARTS]
                    for k in range(OUT_DIM):
                        acc = xs[0] * wvec[k][0]
                        for i in range(1, 7):
                            acc = acc + xs[i] * wvec[k][i]
                        cum = plsc.cumsum(acc)
                        addr = jnp.full((16,), k * CH, jnp.int32) + (t0 + j)
                        plsc.store_scatter(lgbuf, [addr], cum, mask=lane15)
                l0 = lgbuf[pl.ds(t0, 16)] + bvec[0]
                l1 = lgbuf[pl.ds(CH + t0, 16)] + bvec[1]
                l2 = lgbuf[pl.ds(2 * CH + t0, 16)] + bvec[2]
                m = jnp.maximum(jnp.maximum(l0, l1), l2)
                e0 = jnp.exp(l0 - m)
                e1 = jnp.exp(l1 - m)
                e2 = jnp.exp(l2 - m)
                r = 1.0 / (e0 + e1 + e2)
                for k, ek in enumerate((e0, e1, e2)):
                    plsc.store_scatter(
                        obuf, [iota + t0, jnp.full((16,), k, jnp.int32)], ek * r
                    )
                return carry2

            lax.fori_loop(0, NGRP, group, 0)
            pltpu.sync_copy(
                obuf, out_hbm.at[pl.ds(base + c * CH, CH), :]
            )

            @pl.when(c + 2 < NCH)
            def _():
                start_in(c + 2, rb)
        return carry

    lax.fori_loop(0, NCH // 2, chunk_pair, 0)


def kernel(payload, aux, W, b):
    wv = _build_wv(W)
    bb = jnp.broadcast_to(b[:, None], (OUT_DIM, 16))
    mesh = plsc.VectorSubcoreMesh(core_axis_name="c", subcore_axis_name="s")
    flat = pl.kernel(
        _sc_body,
        out_type=jax.ShapeDtypeStruct((N_TOKENS, OUT_DIM), jnp.float32),
        mesh=mesh,
        compiler_params=pltpu.CompilerParams(
            needs_layout_passes=False,
            use_tc_tiling_on_sc=True,
        ),
        scratch_types=[
            pltpu.VMEM((CH, TOKEN_DIM), jnp.float32),
            pltpu.VMEM((CH, TOKEN_DIM), jnp.float32),
            pltpu.VMEM((OUT_DIM * CH,), jnp.float32),
            pltpu.VMEM((CH, OUT_DIM), jnp.float32),
            pltpu.VMEM((OUT_DIM * 7, 16), jnp.float32),
            pltpu.VMEM((OUT_DIM, 16), jnp.float32),
            pltpu.SemaphoreType.DMA((2,)),
        ],
    )(payload, wv, bb)
    return flat


# speedup vs baseline: 1.5016x; 1.2310x over previous
import jax
import jax.numpy as jnp
from jax import lax
from jax.experimental import pallas as pl
from jax.experimental.pallas import tpu as pltpu
from jax.experimental.pallas import tpu_sc as plsc

N_TOKENS = 262144
TOKEN_DIM = 100
ROW_PAD = 128
OUT_DIM = 3
NW = 32
TOK_PER_W = N_TOKENS // NW
CH = 256
NCH = TOK_PER_W // CH
NGRP = CH // 16
SLICE_STARTS = (0, 16, 32, 48, 64, 80, 84)


def _build_wv(W):
    W_ext = jnp.zeros((OUT_DIM, 112), W.dtype).at[:, :TOKEN_DIM].set(W)
    rows = []
    for k in range(OUT_DIM):
        for i, st in enumerate(SLICE_STARTS):
            v = lax.dynamic_slice(W_ext[k], (st,), (16,))
            if i == 6:
                v = v * (jnp.arange(16) >= 12).astype(W.dtype)
            rows.append(v)
    return jnp.stack(rows)


def _sc_body(p_hbm, wv_hbm, bb_hbm, out_hbm, xbuf0, xbuf1, obuf, wv_v, bb_v, in_sem):
    xbufs = (xbuf0, xbuf1)
    wid = lax.axis_index("s") * 2 + lax.axis_index("c")
    base = wid * TOK_PER_W

    pltpu.sync_copy(wv_hbm, wv_v)
    pltpu.sync_copy(bb_hbm, bb_v)
    wvec = [[wv_v[k * 7 + i, :] for i in range(7)] for k in range(OUT_DIM)]
    bvec = [bb_v[k, :] for k in range(OUT_DIM)]
    iota = lax.iota(jnp.int32, 16)
    ix = {s: iota ^ s for s in (1, 2, 4, 8)}
    msk = {s: (iota & s) == 0 for s in (1, 2, 4)}
    low8 = iota < 8

    def lane_shuf(v, s):
        return v.at[ix[s]].get(mode="promise_in_bounds")

    def half_tree(vs):
        for s in (1, 2, 4):
            vs = [
                jnp.where(
                    msk[s],
                    vs[2 * p] + lane_shuf(vs[2 * p], s),
                    vs[2 * p + 1] + lane_shuf(vs[2 * p + 1], s),
                )
                for p in range(len(vs) // 2)
            ]
        z = vs[0]
        return z + lane_shuf(z, 8)

    def start_in(c, rb):
        pltpu.make_async_copy(
            p_hbm.at[pl.ds(base + c * CH, CH), :],
            xbufs[rb],
            in_sem.at[rb],
        ).start()

    for rb in range(2):
        start_in(rb, rb)

    def chunk_pair(c2, carry):
        for rb in range(2):
            c = c2 * 2 + rb
            xb = xbufs[rb]
            pltpu.make_async_copy(
                p_hbm.at[pl.ds(base + c * CH, CH), :],
                xb,
                in_sem.at[rb],
            ).wait()

            def group(g, carry2):
                t0 = g * 16
                halves = []
                for half in range(2):
                    accs = [[], [], []]
                    for j in range(8):
                        t = t0 + half * 8 + j
                        xs = [xb[t, pl.ds(st, 16)] for st in SLICE_STARTS]
                        for k in range(OUT_DIM):
                            a = xs[0] * wvec[k][0]
                            for i in range(1, 7):
                                a = a + xs[i] * wvec[k][i]
                            accs[k].append(a)
                    halves.append([half_tree(accs[k]) for k in range(OUT_DIM)])
                l0 = jnp.where(low8, halves[0][0], halves[1][0]) + bvec[0]
                l1 = jnp.where(low8, halves[0][1], halves[1][1]) + bvec[1]
                l2 = jnp.where(low8, halves[0][2], halves[1][2]) + bvec[2]
                m = jnp.maximum(jnp.maximum(l0, l1), l2)
                e0 = jnp.exp(l0 - m)
                e1 = jnp.exp(l1 - m)
                e2 = jnp.exp(l2 - m)
                r = 1.0 / (e0 + e1 + e2)
                for k, ek in enumerate((e0, e1, e2)):
                    plsc.store_scatter(
                        obuf, [iota + t0, jnp.full((16,), k, jnp.int32)], ek * r
                    )
                return carry2

            lax.fori_loop(0, NGRP, group, 0)
            pltpu.sync_copy(
                obuf, out_hbm.at[pl.ds(base + c * CH, CH), :]
            )

            @pl.when(c + 2 < NCH)
            def _():
                start_in(c + 2, rb)
        return carry

    lax.fori_loop(0, NCH // 2, chunk_pair, 0)


def kernel(payload, aux, W, b):
    wv = _build_wv(W)
    bb = jnp.broadcast_to(b[:, None], (OUT_DIM, 16))
    mesh = plsc.VectorSubcoreMesh(core_axis_name="c", subcore_axis_name="s")
    flat = pl.kernel(
        _sc_body,
        out_type=jax.ShapeDtypeStruct((N_TOKENS, OUT_DIM), jnp.float32),
        mesh=mesh,
        compiler_params=pltpu.CompilerParams(
            needs_layout_passes=False,
            use_tc_tiling_on_sc=True,
        ),
        scratch_types=[
            pltpu.VMEM((CH, TOKEN_DIM), jnp.float32),
            pltpu.VMEM((CH, TOKEN_DIM), jnp.float32),
            pltpu.VMEM((CH, OUT_DIM), jnp.float32),
            pltpu.VMEM((OUT_DIM * 7, 16), jnp.float32),
            pltpu.VMEM((OUT_DIM, 16), jnp.float32),
            pltpu.SemaphoreType.DMA((2,)),
        ],
    )(payload, wv, bb)
    return flat
